# Initial kernel scaffold; baseline (speedup 1.0000x reference)
import jax
import jax.numpy as jnp
from jax.experimental import pallas as pl


def kernel(flat_points, flat_features, segment_ids):
    raise NotImplementedError("write your pallas kernel here")


# speedup vs baseline: 4.0101x; 4.0101x over previous
import functools

import jax
import jax.numpy as jnp
from jax import lax
from jax.experimental import pallas as pl
from jax.experimental.pallas import tpu as pltpu
from jax.experimental.pallas import tpu_sc as plsc

B = 16
T = 32768
F = 128
NC = 2
NS = 16
NW = NC * NS
ROWS_PER_W = T // NW
CH = 128
NCH = ROWS_PER_W // CH
CNT_W = 16


def _seg_pool_body(feat_hbm, ids_hbm, sums_hbm,
                   ids_v, feat_v, zf_v, acc_sh):
    cid = lax.axis_index("c")
    sid = lax.axis_index("s")
    wid = cid * NS + sid
    base = wid * ROWS_PER_W

    zero = jnp.zeros((16,), dtype=jnp.float32)

    @pl.when(sid == 0)
    def _init():
        for i in range(B):
            for j in range(F // 16):
                zf_v[i, pl.ds(j * 16, 16)] = zero
        pltpu.sync_copy(zf_v, acc_sh)

    plsc.subcore_barrier()

    pltpu.sync_copy(ids_hbm.at[pl.ds(wid * NCH, NCH)], ids_v)
    for j in range(NCH):
        pltpu.sync_copy(feat_hbm.at[pl.ds(base + j * CH, CH)], feat_v)
        pltpu.sync_copy(feat_v, acc_sh.at[ids_v.at[j]], add=True)

    plsc.subcore_barrier()

    @pl.when(sid == 0)
    def _writeback():
        pltpu.sync_copy(acc_sh, sums_hbm.at[cid])


_seg_pool = pl.kernel(
    _seg_pool_body,
    out_type=jax.ShapeDtypeStruct((NC, B, F), jnp.float32),
    mesh=plsc.VectorSubcoreMesh(core_axis_name="c", subcore_axis_name="s"),
    scratch_types=[
        pltpu.VMEM((NCH, CH), jnp.int32),
        pltpu.VMEM((CH, F), jnp.float32),
        pltpu.VMEM((B, F), jnp.float32),
        pltpu.VMEM_SHARED((B, F), jnp.float32),
    ],
)


def _combine_body(sums_ref, ids_ref, out_ref):
    s = sums_ref[0] + sums_ref[1]
    ids = ids_ref[...]
    for b in range(B):
        c_b = jnp.sum((ids == b).astype(jnp.float32))
        out_ref[b:b + 1, :] = s[b:b + 1, :] / jnp.maximum(c_b, 1.0)


_combine = pl.pallas_call(
    _combine_body,
    out_shape=jax.ShapeDtypeStruct((B, F), jnp.float32),
)


@jax.jit
def kernel(flat_points, flat_features, segment_ids):
    ids2d = segment_ids.astype(jnp.int32).reshape(T // CH, CH)
    sums = _seg_pool(flat_features, ids2d)
    pooled = _combine(sums, ids2d)
    return (flat_points, pooled)


# speedup vs baseline: 4.6043x; 1.1482x over previous
import functools

import jax
import jax.numpy as jnp
from jax import lax
from jax.experimental import pallas as pl
from jax.experimental.pallas import tpu as pltpu
from jax.experimental.pallas import tpu_sc as plsc

B = 16
T = 32768
F = 128
NC = 2
NS = 16
NW = NC * NS
ROWS_PER_W = T // NW
CH = 128
NCH = ROWS_PER_W // CH
CNT_W = 16


def _seg_pool_body(feat_hbm, ids_hbm, sums_hbm,
                   ids_v, feat_v0, feat_v1, zf_v, acc_sh,
                   gsem0, gsem1, ssem0, ssem1):
    cid = lax.axis_index("c")
    sid = lax.axis_index("s")
    wid = cid * NS + sid
    base = wid * ROWS_PER_W

    bufs = (feat_v0, feat_v1)
    gsems = (gsem0, gsem1)
    ssems = (ssem0, ssem1)

    pltpu.sync_copy(ids_hbm.at[pl.ds(wid * NCH, NCH)], ids_v)
    g_desc = [None, None]
    s_desc = [None, None]
    g_desc[0] = pltpu.async_copy(feat_hbm.at[pl.ds(base, CH)], bufs[0], gsems[0])

    zero = jnp.zeros((16,), dtype=jnp.float32)

    @pl.when(sid == 0)
    def _init():
        for i in range(B):
            for j in range(F // 16):
                zf_v[i, pl.ds(j * 16, 16)] = zero
        pltpu.sync_copy(zf_v, acc_sh)

    plsc.subcore_barrier()

    for j in range(NCH):
        b = j % 2
        g_desc[b].wait()
        if j + 1 < NCH:
            nb = 1 - b
            if s_desc[nb] is not None:
                s_desc[nb].wait()
            g_desc[nb] = pltpu.async_copy(
                feat_hbm.at[pl.ds(base + (j + 1) * CH, CH)], bufs[nb], gsems[nb])
        s_desc[b] = pltpu.async_copy(
            bufs[b], acc_sh.at[ids_v.at[j]], ssems[b], add=True)
    s_desc[0].wait()
    s_desc[1].wait()

    plsc.subcore_barrier()

    @pl.when(sid == 0)
    def _writeback():
        pltpu.sync_copy(acc_sh, sums_hbm.at[cid])


_seg_pool = pl.kernel(
    _seg_pool_body,
    out_type=jax.ShapeDtypeStruct((NC, B, F), jnp.float32),
    mesh=plsc.VectorSubcoreMesh(core_axis_name="c", subcore_axis_name="s"),
    scratch_types=[
        pltpu.VMEM((NCH, CH), jnp.int32),
        pltpu.VMEM((CH, F), jnp.float32),
        pltpu.VMEM((CH, F), jnp.float32),
        pltpu.VMEM((B, F), jnp.float32),
        pltpu.VMEM_SHARED((B, F), jnp.float32),
        pltpu.SemaphoreType.DMA,
        pltpu.SemaphoreType.DMA,
        pltpu.SemaphoreType.DMA,
        pltpu.SemaphoreType.DMA,
    ],
)


def _counts_body(ids_ref, cnt_ref):
    ids = ids_ref[...]
    cols = [jnp.sum((ids == b).astype(jnp.float32)).reshape(1, 1)
            for b in range(B)]
    cnt_ref[...] = jnp.concatenate(cols, axis=1)


_counts = pl.pallas_call(
    _counts_body,
    out_shape=jax.ShapeDtypeStruct((1, B), jnp.float32),
)


def _combine_body(sums_ref, cnt_ref, out_ref):
    s = sums_ref[0] + sums_ref[1]
    c = cnt_ref[0, :][:, None]
    out_ref[...] = s / jnp.maximum(c, 1.0)


_combine = pl.pallas_call(
    _combine_body,
    out_shape=jax.ShapeDtypeStruct((B, F), jnp.float32),
)


@jax.jit
def kernel(flat_points, flat_features, segment_ids):
    ids2d = segment_ids.astype(jnp.int32).reshape(T // CH, CH)
    sums = _seg_pool(flat_features, ids2d)
    cnt = _counts(ids2d)
    pooled = _combine(sums, cnt)
    return (flat_points, pooled)
